# baseline (speedup 1.0000x reference)
import functools

import jax
import jax.numpy as jnp
from jax import lax
from jax.experimental import pallas as pl
from jax.experimental.pallas import tpu as pltpu
from jax.experimental.pallas import tpu_sc as plsc

_K = 64
_NR = 2
_GW = _NR * _K
_B = 16384
_NW = 32
_L = 16
_EPS = 1e-40
_LN2 = 0.6931471805599453

_SC_ROWS = 4096
_TC_ROWS = _B - _SC_ROWS
_RPW = _SC_ROWS // _NW
_GROUPS = _RPW // _L
_TC_BLK = 512


def _soft_log(x):
    bits = lax.bitcast_convert_type(x, jnp.int32)
    e = (bits >> 23) - 127
    m = lax.bitcast_convert_type(
        (bits & 0x007FFFFF) | 0x3F800000, jnp.float32)
    big = m > 1.4142135
    m = jnp.where(big, m * 0.5, m)
    ef = e.astype(jnp.float32)
    ef = jnp.where(big, ef + 1.0, ef)
    r = (m - 1.0) / (m + 1.0)
    r2 = r * r
    p = 2.0 * r * (1.0 + r2 * (1.0 / 3.0 + r2 * (0.2 + r2 * (1.0 / 7.0))))
    return ef * _LN2 + p


def _rule_log_term(t):
    s = 1.0 / (1.0 + jnp.exp(-t))
    return _soft_log(1.0 - s + _EPS)


def _tree_sum(vals):
    while len(vals) > 1:
        vals = [a + b for a, b in zip(vals[::2], vals[1::2])]
    return vals[0]


@functools.partial(
    pl.kernel,
    out_type=jax.ShapeDtypeStruct((_SC_ROWS,), jnp.float32),
    mesh=plsc.VectorSubcoreMesh(core_axis_name="c", subcore_axis_name="s"),
    scratch_types=[
        pltpu.VMEM((_RPW, _GW), jnp.float32),
        pltpu.VMEM((_RPW, _K), jnp.float32),
        pltpu.VMEM((_RPW,), jnp.float32),
        pltpu.SemaphoreType.DMA,
    ],
    compiler_params=pltpu.CompilerParams(
        needs_layout_passes=False,
        disable_bounds_checks=True,
        skip_device_barrier=True,
        use_tc_tiling_on_sc=False,
    ),
)
def _sc_fwd(gu_hbm, gi_hbm, out_hbm, gu_v, gi_v, out_v, dsem):
    wid = lax.axis_index("s") * 2 + lax.axis_index("c")
    base = wid * _RPW
    hu = pltpu.async_copy(gu_hbm.at[pl.ds(base, _RPW)], gu_v, dsem)
    hi = pltpu.async_copy(gi_hbm.at[pl.ds(base, _RPW)], gi_v, dsem)
    hu.wait()
    hi.wait()

    def group(g):
        lane = lax.iota(jnp.int32, _L)
        rows = lane + g * _L
        p0, p1 = [], []
        for j in range(_K):
            o = (lane + j) & (_K - 1)
            giv = plsc.load_gather(gi_v, [rows, o])
            u0 = plsc.load_gather(gu_v, [rows, o])
            u1 = plsc.load_gather(gu_v, [rows, o + _K])
            p0.append(u0 * giv)
            p1.append(u1 * giv)
        sum_log = (_rule_log_term(_tree_sum(p0))
                   + _rule_log_term(_tree_sum(p1)))
        out_v[pl.ds(g * _L, _L)] = 1.0 - (-1.0 / (-1.0 + sum_log))

    plsc.parallel_loop(0, _GROUPS, 1, unroll=2)(group)
    pltpu.sync_copy(out_v, out_hbm.at[pl.ds(base, _RPW)])


def _tc_body(gu_ref, gi_ref, out_ref):
    gu = gu_ref[...]
    gi = gi_ref[...]
    t0 = jnp.sum(gu[:, :_K] * gi, axis=1, keepdims=True)
    t1 = jnp.sum(gu[:, _K:] * gi, axis=1, keepdims=True)
    s0 = 1.0 / (1.0 + jnp.exp(-t0))
    s1 = 1.0 / (1.0 + jnp.exp(-t1))
    sum_log = jnp.log(1.0 - s0 + _EPS) + jnp.log(1.0 - s1 + _EPS)
    out_ref[...] = 1.0 - (-1.0 / (-1.0 + sum_log))


_tc_fwd = pl.pallas_call(
    _tc_body,
    out_shape=jax.ShapeDtypeStruct((_TC_ROWS, 1), jnp.float32),
    grid=(_TC_ROWS // _TC_BLK,),
    in_specs=[
        pl.BlockSpec((_TC_BLK, _GW), lambda i: (i, 0)),
        pl.BlockSpec((_TC_BLK, _K), lambda i: (i, 0)),
    ],
    out_specs=pl.BlockSpec((_TC_BLK, 1), lambda i: (i, 0)),
)


def kernel(gu, gi):
    xui_sc = _sc_fwd(gu[_TC_ROWS:], gi[_TC_ROWS:])
    xui_tc = _tc_fwd(gu, gi)[:, 0]
    return jnp.concatenate([xui_tc, xui_sc])


# speedup vs baseline: 1.0118x; 1.0118x over previous
import functools

import jax
import jax.numpy as jnp
from jax import lax
from jax.experimental import pallas as pl
from jax.experimental.pallas import tpu as pltpu
from jax.experimental.pallas import tpu_sc as plsc

_K = 64
_NR = 2
_GW = _NR * _K
_B = 16384
_NW = 32
_L = 16
_EPS = 1e-40
_LN2 = 0.6931471805599453

_SC_ROWS = 4096
_TC_ROWS = _B - _SC_ROWS
_RPW = _SC_ROWS // _NW
_GROUPS = _RPW // _L
_TC_BLK = 512


def _soft_log(x):
    bits = lax.bitcast_convert_type(x, jnp.int32)
    e = (bits >> 23) - 127
    m = lax.bitcast_convert_type(
        (bits & 0x007FFFFF) | 0x3F800000, jnp.float32)
    big = m > 1.4142135
    m = jnp.where(big, m * 0.5, m)
    ef = e.astype(jnp.float32)
    ef = jnp.where(big, ef + 1.0, ef)
    r = (m - 1.0) / (m + 1.0)
    r2 = r * r
    p = 2.0 * r * (1.0 + r2 * (1.0 / 3.0 + r2 * (0.2 + r2 * (1.0 / 7.0))))
    return ef * _LN2 + p


def _rule_log_term(t):
    s = 1.0 / (1.0 + jnp.exp(-t))
    return _soft_log(1.0 - s + _EPS)


def _tree_sum(vals):
    while len(vals) > 1:
        vals = [a + b for a, b in zip(vals[::2], vals[1::2])]
    return vals[0]


@functools.partial(
    pl.kernel,
    out_type=jax.ShapeDtypeStruct((_SC_ROWS,), jnp.float32),
    mesh=plsc.VectorSubcoreMesh(core_axis_name="c", subcore_axis_name="s"),
    scratch_types=[
        pltpu.VMEM((_RPW, _GW), jnp.float32),
        pltpu.VMEM((_RPW, _K), jnp.float32),
        pltpu.VMEM((_RPW,), jnp.float32),
        pltpu.SemaphoreType.DMA,
    ],
    compiler_params=pltpu.CompilerParams(
        needs_layout_passes=False,
        disable_bounds_checks=True,
        skip_device_barrier=True,
        use_tc_tiling_on_sc=False,
    ),
)
def _sc_fwd(gu_hbm, gi_hbm, out_hbm, gu_v, gi_v, out_v, dsem):
    wid = lax.axis_index("s") * 2 + lax.axis_index("c")
    base = wid * _RPW
    hu = pltpu.async_copy(gu_hbm.at[pl.ds(base, _RPW)], gu_v, dsem)
    hi = pltpu.async_copy(gi_hbm.at[pl.ds(base, _RPW)], gi_v, dsem)
    hu.wait()
    hi.wait()

    def group(g):
        lane = lax.iota(jnp.int32, _L)
        rows = lane + g * _L
        p0, p1 = [], []
        for j in range(_K):
            o = (lane + j) & (_K - 1)
            giv = plsc.load_gather(gi_v, [rows, o])
            u0 = plsc.load_gather(gu_v, [rows, o])
            u1 = plsc.load_gather(gu_v, [rows, o + _K])
            p0.append(u0 * giv)
            p1.append(u1 * giv)
        sum_log = (_rule_log_term(_tree_sum(p0))
                   + _rule_log_term(_tree_sum(p1)))
        out_v[pl.ds(g * _L, _L)] = 1.0 - (-1.0 / (-1.0 + sum_log))

    plsc.parallel_loop(0, _GROUPS, 1, unroll=2)(group)
    pltpu.sync_copy(out_v, out_hbm.at[pl.ds(base, _RPW)])


def _tc_body(gu_ref, gi_ref, out_ref):
    gu = gu_ref[...]
    gi = gi_ref[...]
    prod = gu * jnp.concatenate([gi, gi], axis=1)
    sel = (lax.broadcasted_iota(jnp.int32, (_GW, _NR), 0) // _K
           == lax.broadcasted_iota(jnp.int32, (_GW, _NR), 1)
           ).astype(jnp.float32)
    t = jax.lax.dot_general(prod, sel, (((1,), (0,)), ((), ())),
                            preferred_element_type=jnp.float32)
    s = 1.0 / (1.0 + jnp.exp(-t))
    sum_log = jnp.sum(jnp.log(1.0 - s + _EPS), axis=1, keepdims=True)
    out_ref[...] = 1.0 - (-1.0 / (-1.0 + sum_log))


_tc_fwd = pl.pallas_call(
    _tc_body,
    out_shape=jax.ShapeDtypeStruct((_TC_ROWS, 1), jnp.float32),
    grid=(_TC_ROWS // _TC_BLK,),
    in_specs=[
        pl.BlockSpec((_TC_BLK, _GW), lambda i: (i, 0)),
        pl.BlockSpec((_TC_BLK, _K), lambda i: (i, 0)),
    ],
    out_specs=pl.BlockSpec((_TC_BLK, 1), lambda i: (i, 0)),
)


def kernel(gu, gi):
    xui_sc = _sc_fwd(gu[_TC_ROWS:], gi[_TC_ROWS:])
    xui_tc = _tc_fwd(gu, gi)[:, 0]
    return jnp.concatenate([xui_tc, xui_sc])


# speedup vs baseline: 1.3066x; 1.2913x over previous
import functools

import jax
import jax.numpy as jnp
from jax import lax
from jax.experimental import pallas as pl
from jax.experimental.pallas import tpu as pltpu
from jax.experimental.pallas import tpu_sc as plsc

_K = 64
_NR = 2
_GW = _NR * _K
_B = 16384
_NW = 32
_L = 16
_EPS = 1e-40
_LN2 = 0.6931471805599453

_SC_ROWS = 4096
_TC_ROWS = _B - _SC_ROWS
_RPW = _SC_ROWS // _NW
_GROUPS = _RPW // _L
_TC_BLK = 1024


def _soft_log(x):
    bits = lax.bitcast_convert_type(x, jnp.int32)
    e = (bits >> 23) - 127
    m = lax.bitcast_convert_type(
        (bits & 0x007FFFFF) | 0x3F800000, jnp.float32)
    big = m > 1.4142135
    m = jnp.where(big, m * 0.5, m)
    ef = e.astype(jnp.float32)
    ef = jnp.where(big, ef + 1.0, ef)
    r = (m - 1.0) / (m + 1.0)
    r2 = r * r
    p = 2.0 * r * (1.0 + r2 * (1.0 / 3.0 + r2 * (0.2 + r2 * (1.0 / 7.0))))
    return ef * _LN2 + p


def _rule_log_term(t):
    s = 1.0 / (1.0 + jnp.exp(-t))
    return _soft_log(1.0 - s + _EPS)


def _tree_sum(vals):
    while len(vals) > 1:
        vals = [a + b for a, b in zip(vals[::2], vals[1::2])]
    return vals[0]


@functools.partial(
    pl.kernel,
    out_type=jax.ShapeDtypeStruct((_SC_ROWS,), jnp.float32),
    mesh=plsc.VectorSubcoreMesh(core_axis_name="c", subcore_axis_name="s"),
    scratch_types=[
        pltpu.VMEM((_RPW, _GW), jnp.float32),
        pltpu.VMEM((_RPW, _K), jnp.float32),
        pltpu.VMEM((_RPW,), jnp.float32),
        pltpu.SemaphoreType.DMA,
    ],
    compiler_params=pltpu.CompilerParams(
        needs_layout_passes=False,
        disable_bounds_checks=True,
        skip_device_barrier=True,
        use_tc_tiling_on_sc=False,
    ),
)
def _sc_fwd(gu_hbm, gi_hbm, out_hbm, gu_v, gi_v, out_v, dsem):
    wid = lax.axis_index("s") * 2 + lax.axis_index("c")
    base = wid * _RPW
    hu = pltpu.async_copy(gu_hbm.at[pl.ds(base, _RPW)], gu_v, dsem)
    hi = pltpu.async_copy(gi_hbm.at[pl.ds(base, _RPW)], gi_v, dsem)
    hu.wait()
    hi.wait()

    def group(g):
        lane = lax.iota(jnp.int32, _L)
        rows = lane + g * _L
        p0, p1 = [], []
        for j in range(_K):
            o = (lane + j) & (_K - 1)
            giv = plsc.load_gather(gi_v, [rows, o])
            u0 = plsc.load_gather(gu_v, [rows, o])
            u1 = plsc.load_gather(gu_v, [rows, o + _K])
            p0.append(u0 * giv)
            p1.append(u1 * giv)
        sum_log = (_rule_log_term(_tree_sum(p0))
                   + _rule_log_term(_tree_sum(p1)))
        out_v[pl.ds(g * _L, _L)] = 1.0 - (-1.0 / (-1.0 + sum_log))

    plsc.parallel_loop(0, _GROUPS, 1, unroll=2)(group)
    pltpu.sync_copy(out_v, out_hbm.at[pl.ds(base, _RPW)])


def _tc_body(gu_ref, gi_ref, out_ref):
    gu = gu_ref[...]
    gi = gi_ref[...]
    prod = gu * jnp.concatenate([gi, gi], axis=1)
    sel = (lax.broadcasted_iota(jnp.int32, (_NR, _GW), 0)
           == lax.broadcasted_iota(jnp.int32, (_NR, _GW), 1) // _K
           ).astype(jnp.float32)
    t = jax.lax.dot_general(sel, prod, (((1,), (1,)), ((), ())),
                            preferred_element_type=jnp.float32)
    s = 1.0 / (1.0 + jnp.exp(-t))
    sum_log = jnp.sum(jnp.log(1.0 - s + _EPS), axis=0, keepdims=True)
    out_ref[...] = 1.0 - (-1.0 / (-1.0 + sum_log))


_tc_fwd = pl.pallas_call(
    _tc_body,
    out_shape=jax.ShapeDtypeStruct((1, _TC_ROWS), jnp.float32),
    grid=(_TC_ROWS // _TC_BLK,),
    in_specs=[
        pl.BlockSpec((_TC_BLK, _GW), lambda i: (i, 0)),
        pl.BlockSpec((_TC_BLK, _K), lambda i: (i, 0)),
    ],
    out_specs=pl.BlockSpec((1, _TC_BLK), lambda i: (0, i)),
)


def kernel(gu, gi):
    xui_sc = _sc_fwd(gu[_TC_ROWS:], gi[_TC_ROWS:])
    xui_tc = _tc_fwd(gu, gi).reshape(_TC_ROWS)
    return jnp.concatenate([xui_tc, xui_sc])


# speedup vs baseline: 1.5522x; 1.1880x over previous
import functools

import jax
import jax.numpy as jnp
from jax import lax
from jax.experimental import pallas as pl
from jax.experimental.pallas import tpu as pltpu
from jax.experimental.pallas import tpu_sc as plsc

_K = 64
_NR = 2
_GW = _NR * _K
_B = 16384
_NW = 32
_L = 16
_EPS = 1e-40
_LN2 = 0.6931471805599453

_SC_ROWS = 4096
_TC_ROWS = _B - _SC_ROWS
_RPW = _SC_ROWS // _NW
_GROUPS = _RPW // _L
_TC_BLK = 1024


def _soft_log(x):
    bits = lax.bitcast_convert_type(x, jnp.int32)
    e = (bits >> 23) - 127
    m = lax.bitcast_convert_type(
        (bits & 0x007FFFFF) | 0x3F800000, jnp.float32)
    big = m > 1.4142135
    m = jnp.where(big, m * 0.5, m)
    ef = e.astype(jnp.float32)
    ef = jnp.where(big, ef + 1.0, ef)
    r = (m - 1.0) / (m + 1.0)
    r2 = r * r
    p = 2.0 * r * (1.0 + r2 * (1.0 / 3.0 + r2 * (0.2 + r2 * (1.0 / 7.0))))
    return ef * _LN2 + p


def _rule_log_term(t):
    s = 1.0 / (1.0 + jnp.exp(-t))
    return _soft_log(1.0 - s + _EPS)


def _tree_sum(vals):
    while len(vals) > 1:
        vals = [a + b for a, b in zip(vals[::2], vals[1::2])]
    return vals[0]


@functools.partial(
    pl.kernel,
    out_type=jax.ShapeDtypeStruct((_SC_ROWS,), jnp.float32),
    mesh=plsc.VectorSubcoreMesh(core_axis_name="c", subcore_axis_name="s"),
    scratch_types=[
        pltpu.VMEM((_RPW, _GW), jnp.float32),
        pltpu.VMEM((_RPW, _K), jnp.float32),
        pltpu.VMEM((_RPW,), jnp.float32),
        pltpu.SemaphoreType.DMA,
    ],
    compiler_params=pltpu.CompilerParams(
        needs_layout_passes=False,
        disable_bounds_checks=True,
        skip_device_barrier=True,
        use_tc_tiling_on_sc=False,
    ),
)
def _sc_fwd(gu_hbm, gi_hbm, out_hbm, gu_v, gi_v, out_v, dsem):
    wid = lax.axis_index("s") * 2 + lax.axis_index("c")
    base = wid * _RPW
    hu = pltpu.async_copy(gu_hbm.at[pl.ds(base, _RPW)], gu_v, dsem)
    hi = pltpu.async_copy(gi_hbm.at[pl.ds(base, _RPW)], gi_v, dsem)
    hu.wait()
    hi.wait()

    def group(g):
        lane = lax.iota(jnp.int32, _L)
        rows = lane + g * _L
        p0, p1 = [], []
        for j in range(_K):
            o = (lane + j) & (_K - 1)
            giv = plsc.load_gather(gi_v, [rows, o])
            u0 = plsc.load_gather(gu_v, [rows, o])
            u1 = plsc.load_gather(gu_v, [rows, o + _K])
            p0.append(u0 * giv)
            p1.append(u1 * giv)
        sum_log = (_rule_log_term(_tree_sum(p0))
                   + _rule_log_term(_tree_sum(p1)))
        out_v[pl.ds(g * _L, _L)] = 1.0 - (-1.0 / (-1.0 + sum_log))

    plsc.parallel_loop(0, _GROUPS, 1, unroll=2)(group)
    pltpu.sync_copy(out_v, out_hbm.at[pl.ds(base, _RPW)])


def _tc_body(gu_ref, git_ref, out_ref):
    gu = gu_ref[...]
    gi = lax.transpose(git_ref[...], (1, 0))
    prod = gu * jnp.concatenate([gi, gi], axis=1)
    sel = (lax.broadcasted_iota(jnp.int32, (_NR, _GW), 0)
           == lax.broadcasted_iota(jnp.int32, (_NR, _GW), 1) // _K
           ).astype(jnp.float32)
    t = jax.lax.dot_general(sel, prod, (((1,), (1,)), ((), ())),
                            preferred_element_type=jnp.float32)
    s = 1.0 / (1.0 + jnp.exp(-t))
    sum_log = jnp.sum(jnp.log(1.0 - s + _EPS), axis=0, keepdims=True)
    out_ref[...] = 1.0 - (-1.0 / (-1.0 + sum_log))


_tc_fwd = pl.pallas_call(
    _tc_body,
    out_shape=jax.ShapeDtypeStruct((1, _TC_ROWS), jnp.float32),
    grid=(_TC_ROWS // _TC_BLK,),
    in_specs=[
        pl.BlockSpec((_TC_BLK, _GW), lambda i: (i, 0)),
        pl.BlockSpec((_K, _TC_BLK), lambda i: (0, i)),
    ],
    out_specs=pl.BlockSpec((1, _TC_BLK), lambda i: (0, i)),
)


def kernel(gu, gi):
    xui_sc = _sc_fwd(gu[_TC_ROWS:], gi[_TC_ROWS:])
    xui_tc = _tc_fwd(gu, gi.T).reshape(_TC_ROWS)
    return jnp.concatenate([xui_tc, xui_sc])


# speedup vs baseline: 1.7152x; 1.1050x over previous
import functools

import jax
import jax.numpy as jnp
from jax import lax
from jax.experimental import pallas as pl
from jax.experimental.pallas import tpu as pltpu
from jax.experimental.pallas import tpu_sc as plsc

_K = 64
_NR = 2
_GW = _NR * _K
_B = 16384
_NW = 32
_L = 16
_EPS = 1e-40
_LN2 = 0.6931471805599453

_SC_ROWS = 2048
_TC_ROWS = _B - _SC_ROWS
_RPW = _SC_ROWS // _NW
_GROUPS = _RPW // _L
_TC_BLK = 2048


def _soft_log(x):
    bits = lax.bitcast_convert_type(x, jnp.int32)
    e = (bits >> 23) - 127
    m = lax.bitcast_convert_type(
        (bits & 0x007FFFFF) | 0x3F800000, jnp.float32)
    big = m > 1.4142135
    m = jnp.where(big, m * 0.5, m)
    ef = e.astype(jnp.float32)
    ef = jnp.where(big, ef + 1.0, ef)
    r = (m - 1.0) / (m + 1.0)
    r2 = r * r
    p = 2.0 * r * (1.0 + r2 * (1.0 / 3.0 + r2 * (0.2 + r2 * (1.0 / 7.0))))
    return ef * _LN2 + p


def _rule_log_term(t):
    s = 1.0 / (1.0 + jnp.exp(-t))
    return _soft_log(1.0 - s + _EPS)


def _tree_sum(vals):
    while len(vals) > 1:
        vals = [a + b for a, b in zip(vals[::2], vals[1::2])]
    return vals[0]


@functools.partial(
    pl.kernel,
    out_type=jax.ShapeDtypeStruct((_SC_ROWS,), jnp.float32),
    mesh=plsc.VectorSubcoreMesh(core_axis_name="c", subcore_axis_name="s"),
    scratch_types=[
        pltpu.VMEM((_RPW, _GW), jnp.float32),
        pltpu.VMEM((_RPW, _K), jnp.float32),
        pltpu.VMEM((_RPW,), jnp.float32),
        pltpu.SemaphoreType.DMA,
    ],
    compiler_params=pltpu.CompilerParams(
        needs_layout_passes=False,
        disable_bounds_checks=True,
        skip_device_barrier=True,
        use_tc_tiling_on_sc=False,
    ),
)
def _sc_fwd(gu_hbm, gi_hbm, out_hbm, gu_v, gi_v, out_v, dsem):
    wid = lax.axis_index("s") * 2 + lax.axis_index("c")
    base = wid * _RPW
    hu = pltpu.async_copy(gu_hbm.at[pl.ds(base, _RPW)], gu_v, dsem)
    hi = pltpu.async_copy(gi_hbm.at[pl.ds(base, _RPW)], gi_v, dsem)
    hu.wait()
    hi.wait()

    def group(g):
        lane = lax.iota(jnp.int32, _L)
        rows = lane + g * _L
        p0, p1 = [], []
        for j in range(_K):
            o = (lane + j) & (_K - 1)
            giv = plsc.load_gather(gi_v, [rows, o])
            u0 = plsc.load_gather(gu_v, [rows, o])
            u1 = plsc.load_gather(gu_v, [rows, o + _K])
            p0.append(u0 * giv)
            p1.append(u1 * giv)
        sum_log = (_rule_log_term(_tree_sum(p0))
                   + _rule_log_term(_tree_sum(p1)))
        out_v[pl.ds(g * _L, _L)] = 1.0 - (-1.0 / (-1.0 + sum_log))

    plsc.parallel_loop(0, _GROUPS, 1, unroll=2)(group)
    pltpu.sync_copy(out_v, out_hbm.at[pl.ds(base, _RPW)])


def _tc_body(gu_ref, git_ref, out_ref):
    gu = gu_ref[...]
    gi = lax.transpose(git_ref[...], (1, 0))
    prod = gu * jnp.concatenate([gi, gi], axis=1)
    sel = (lax.broadcasted_iota(jnp.int32, (_NR, _GW), 0)
           == lax.broadcasted_iota(jnp.int32, (_NR, _GW), 1) // _K
           ).astype(jnp.float32)
    t = jax.lax.dot_general(sel, prod, (((1,), (1,)), ((), ())),
                            preferred_element_type=jnp.float32)
    s = 1.0 / (1.0 + jnp.exp(-t))
    sum_log = jnp.sum(jnp.log(1.0 - s + _EPS), axis=0, keepdims=True)
    out_ref[...] = 1.0 - (-1.0 / (-1.0 + sum_log))


_tc_fwd = pl.pallas_call(
    _tc_body,
    out_shape=jax.ShapeDtypeStruct((1, _TC_ROWS), jnp.float32),
    grid=(_TC_ROWS // _TC_BLK,),
    in_specs=[
        pl.BlockSpec((_TC_BLK, _GW), lambda i: (i, 0)),
        pl.BlockSpec((_K, _TC_BLK), lambda i: (0, i)),
    ],
    out_specs=pl.BlockSpec((1, _TC_BLK), lambda i: (0, i)),
)


def kernel(gu, gi):
    xui_sc = _sc_fwd(gu[_TC_ROWS:], gi[_TC_ROWS:])
    xui_tc = _tc_fwd(gu, gi.T).reshape(_TC_ROWS)
    return jnp.concatenate([xui_tc, xui_sc])


# speedup vs baseline: 1.8655x; 1.0877x over previous
import functools

import jax
import jax.numpy as jnp
from jax import lax
from jax.experimental import pallas as pl
from jax.experimental.pallas import tpu as pltpu
from jax.experimental.pallas import tpu_sc as plsc

_K = 64
_NR = 2
_GW = _NR * _K
_B = 16384
_NW = 32
_L = 16
_EPS = 1e-40
_LN2 = 0.6931471805599453

_SC_ROWS = 1024
_TC_ROWS = _B - _SC_ROWS
_RPW = _SC_ROWS // _NW
_GROUPS = _RPW // _L
_TC_BLK = 3840


def _soft_log(x):
    bits = lax.bitcast_convert_type(x, jnp.int32)
    e = (bits >> 23) - 127
    m = lax.bitcast_convert_type(
        (bits & 0x007FFFFF) | 0x3F800000, jnp.float32)
    big = m > 1.4142135
    m = jnp.where(big, m * 0.5, m)
    ef = e.astype(jnp.float32)
    ef = jnp.where(big, ef + 1.0, ef)
    r = (m - 1.0) / (m + 1.0)
    r2 = r * r
    p = 2.0 * r * (1.0 + r2 * (1.0 / 3.0 + r2 * (0.2 + r2 * (1.0 / 7.0))))
    return ef * _LN2 + p


def _rule_log_term(t):
    s = 1.0 / (1.0 + jnp.exp(-t))
    return _soft_log(1.0 - s + _EPS)


def _tree_sum(vals):
    while len(vals) > 1:
        vals = [a + b for a, b in zip(vals[::2], vals[1::2])]
    return vals[0]


@functools.partial(
    pl.kernel,
    out_type=jax.ShapeDtypeStruct((_SC_ROWS,), jnp.float32),
    mesh=plsc.VectorSubcoreMesh(core_axis_name="c", subcore_axis_name="s"),
    scratch_types=[
        pltpu.VMEM((_RPW, _GW), jnp.float32),
        pltpu.VMEM((_RPW, _K), jnp.float32),
        pltpu.VMEM((_RPW,), jnp.float32),
        pltpu.SemaphoreType.DMA,
    ],
    compiler_params=pltpu.CompilerParams(
        needs_layout_passes=False,
        disable_bounds_checks=True,
        skip_device_barrier=True,
        use_tc_tiling_on_sc=False,
    ),
)
def _sc_fwd(gu_hbm, gi_hbm, out_hbm, gu_v, gi_v, out_v, dsem):
    wid = lax.axis_index("s") * 2 + lax.axis_index("c")
    base = wid * _RPW
    hu = pltpu.async_copy(gu_hbm.at[pl.ds(base, _RPW)], gu_v, dsem)
    hi = pltpu.async_copy(gi_hbm.at[pl.ds(base, _RPW)], gi_v, dsem)
    hu.wait()
    hi.wait()

    def group(g):
        lane = lax.iota(jnp.int32, _L)
        rows = lane + g * _L
        p0, p1 = [], []
        for j in range(_K):
            o = (lane + j) & (_K - 1)
            giv = plsc.load_gather(gi_v, [rows, o])
            u0 = plsc.load_gather(gu_v, [rows, o])
            u1 = plsc.load_gather(gu_v, [rows, o + _K])
            p0.append(u0 * giv)
            p1.append(u1 * giv)
        sum_log = (_rule_log_term(_tree_sum(p0))
                   + _rule_log_term(_tree_sum(p1)))
        out_v[pl.ds(g * _L, _L)] = 1.0 - (-1.0 / (-1.0 + sum_log))

    plsc.parallel_loop(0, _GROUPS, 1, unroll=2)(group)
    pltpu.sync_copy(out_v, out_hbm.at[pl.ds(base, _RPW)])


def _tc_body(gu_ref, git_ref, out_ref):
    gu = gu_ref[...]
    gi = lax.transpose(git_ref[...], (1, 0))
    prod = gu * jnp.concatenate([gi, gi], axis=1)
    sel = (lax.broadcasted_iota(jnp.int32, (_NR, _GW), 0)
           == lax.broadcasted_iota(jnp.int32, (_NR, _GW), 1) // _K
           ).astype(jnp.float32)
    t = jax.lax.dot_general(sel, prod, (((1,), (1,)), ((), ())),
                            preferred_element_type=jnp.float32)
    s = 1.0 / (1.0 + jnp.exp(-t))
    sum_log = jnp.sum(jnp.log(1.0 - s + _EPS), axis=0, keepdims=True)
    out_ref[...] = 1.0 - (-1.0 / (-1.0 + sum_log))


_tc_fwd = pl.pallas_call(
    _tc_body,
    out_shape=jax.ShapeDtypeStruct((1, _TC_ROWS), jnp.float32),
    grid=(_TC_ROWS // _TC_BLK,),
    in_specs=[
        pl.BlockSpec((_TC_BLK, _GW), lambda i: (i, 0)),
        pl.BlockSpec((_K, _TC_BLK), lambda i: (0, i)),
    ],
    out_specs=pl.BlockSpec((1, _TC_BLK), lambda i: (0, i)),
)


def kernel(gu, gi):
    xui_sc = _sc_fwd(gu[_TC_ROWS:], gi[_TC_ROWS:])
    xui_tc = _tc_fwd(gu, gi.T).reshape(_TC_ROWS)
    return jnp.concatenate([xui_tc, xui_sc])
